# baseline (speedup 1.0000x reference)
import functools

import jax
import jax.numpy as jnp
from jax import lax
from jax.experimental import pallas as pl
from jax.experimental.pallas import tpu as pltpu
from jax.experimental.pallas import tpu_sc as plsc

N = 10000
D = 128
E = 320000
NUM_LAYERS = 5

NC = 2
NS = 16
NW = NC * NS
CH = 128
CHUNKS = 80
E_PAD = NW * CHUNKS * CH
N_ACC = 10112
ROWS_PER_TILE = N_ACC // NS
CNT_W = 16

_mesh = plsc.VectorSubcoreMesh(
    core_axis_name="c", subcore_axis_name="s", num_cores=NC, num_subcores=NS)


NBUF = 2
GIDX = 8
N0 = 160
N1 = 0
C0 = NS * N0


def _agg_body(h_hbm, srcm, dstm, zeros_hbm, p_hbm, acc,
              idx_v, rows0, rows1, isem, g0, g1, s0, s1):
    cid = lax.axis_index("c")
    sid = lax.axis_index("s")
    rows = (rows0, rows1)
    gsem = (g0, g1)
    ssem = (s0, s1)
    my_chunks = jnp.where(cid == 0, N0, N1)
    base_chunk = jnp.where(cid == 0, sid * N0, C0 + sid * N1)
    ngroups = my_chunks // GIDX

    @pl.when(ngroups > 0)
    def _prefetch0():
        pltpu.async_copy(srcm.at[pl.ds(base_chunk, GIDX)], idx_v.at[0, 0], isem)
        pltpu.async_copy(dstm.at[pl.ds(base_chunk, GIDX)], idx_v.at[0, 1], isem)
    pltpu.sync_copy(zeros_hbm.at[pl.ds(sid * ROWS_PER_TILE, ROWS_PER_TILE)],
                    acc.at[pl.ds(sid * ROWS_PER_TILE, ROWS_PER_TILE)])
    plsc.subcore_barrier()

    def group(g, carry):
        pb = g % 2
        pltpu.make_async_copy(srcm.at[pl.ds(base_chunk, GIDX)],
                              idx_v.at[0, 0], isem).wait()
        pltpu.make_async_copy(dstm.at[pl.ds(base_chunk, GIDX)],
                              idx_v.at[0, 1], isem).wait()

        @pl.when(g + 1 < ngroups)
        def _prefetch():
            nb = (g + 1) % 2
            off = base_chunk + (g + 1) * GIDX
            pltpu.async_copy(srcm.at[pl.ds(off, GIDX)], idx_v.at[nb, 0], isem)
            pltpu.async_copy(dstm.at[pl.ds(off, GIDX)], idx_v.at[nb, 1], isem)

        def step(j2, c2):
            base = j2 * NBUF
            gathers = [
                pltpu.async_copy(h_hbm.at[idx_v.at[pb, 0, base + b]],
                                 rows[b], gsem[b])
                for b in range(NBUF)
            ]
            scatters = []
            for b in range(NBUF):
                gathers[b].wait()
                scatters.append(
                    pltpu.async_copy(rows[b],
                                     acc.at[idx_v.at[pb, 1, base + b]],
                                     ssem[b], add=True))
            for b in range(NBUF):
                scatters[b].wait()
            return c2

        lax.fori_loop(0, GIDX // NBUF, step, 0)
        return carry

    lax.fori_loop(0, ngroups, group, 0)

    plsc.subcore_barrier()
    pltpu.sync_copy(acc.at[pl.ds(sid * ROWS_PER_TILE, ROWS_PER_TILE)],
                    p_hbm.at[cid, pl.ds(sid * ROWS_PER_TILE, ROWS_PER_TILE)])


_agg_call = pl.kernel(
    _agg_body,
    out_type=jax.ShapeDtypeStruct((NC, N_ACC, D), jnp.float32),
    mesh=_mesh,
    scratch_types=[
        pltpu.VMEM_SHARED((N_ACC, D), jnp.float32),
        pltpu.VMEM((2, 2, GIDX, CH), jnp.int32),
        pltpu.VMEM((CH, D), jnp.float32),
        pltpu.VMEM((CH, D), jnp.float32),
        pltpu.SemaphoreType.DMA,
        pltpu.SemaphoreType.DMA,
        pltpu.SemaphoreType.DMA,
        pltpu.SemaphoreType.DMA,
        pltpu.SemaphoreType.DMA,
    ],
)


def _cnt_body(dstm, ones_hbm, zeros_hbm, c_hbm, acc, dst_v, ones_v):
    cid = lax.axis_index("c")
    sid = lax.axis_index("s")
    w = sid * NC + cid
    pltpu.sync_copy(zeros_hbm.at[pl.ds(sid * ROWS_PER_TILE, ROWS_PER_TILE)],
                    acc.at[pl.ds(sid * ROWS_PER_TILE, ROWS_PER_TILE)])
    pltpu.sync_copy(ones_hbm, ones_v)
    pltpu.sync_copy(dstm.at[pl.ds(w * CHUNKS, CHUNKS)], dst_v)
    plsc.subcore_barrier()

    def step(j, carry):
        pltpu.sync_copy(ones_v, acc.at[dst_v.at[j]], add=True)
        return carry

    lax.fori_loop(0, CHUNKS, step, 0)
    plsc.subcore_barrier()
    pltpu.sync_copy(acc.at[pl.ds(sid * ROWS_PER_TILE, ROWS_PER_TILE)],
                    c_hbm.at[cid, pl.ds(sid * ROWS_PER_TILE, ROWS_PER_TILE)])


_cnt_call = pl.kernel(
    _cnt_body,
    out_type=jax.ShapeDtypeStruct((NC, N_ACC, D), jnp.float32),
    mesh=_mesh,
    scratch_types=[
        pltpu.VMEM_SHARED((N_ACC, D), jnp.float32),
        pltpu.VMEM((CHUNKS, CH), jnp.int32),
        pltpu.VMEM((CH, D), jnp.float32),
    ],
)


def _narrow_body(c_ref, o_ref):
    o_ref[...] = c_ref[0, :, :CNT_W] + c_ref[1, :, :CNT_W]


def _dense_body(apply_ln, p_ref, cnt_ref, h_ref, wl_ref, bl_ref, wr_ref,
                g_ref, b_ref, o_ref):
    p = p_ref[0] + p_ref[1]
    c = cnt_ref[:, 0:1]
    mean = p / jnp.maximum(c, 1.0)
    out = lax.dot_general(mean, wl_ref[...], (((1,), (1,)), ((), ())),
                          preferred_element_type=jnp.float32)
    out = out + bl_ref[...]
    out = out + lax.dot_general(h_ref[...], wr_ref[...], (((1,), (1,)), ((), ())),
                                preferred_element_type=jnp.float32)
    if apply_ln:
        mu = jnp.mean(out, axis=-1, keepdims=True)
        var = jnp.mean((out - mu) ** 2, axis=-1, keepdims=True)
        out = (out - mu) * lax.rsqrt(var + 1e-5) * g_ref[...] + b_ref[...]
        out = jnp.maximum(out, 0.0)
    o_ref[...] = out


BN = 400


_narrow_call = pl.pallas_call(
    _narrow_body,
    grid=(N_ACC // 632,),
    in_specs=[pl.BlockSpec((NC, 632, D), lambda i: (0, i, 0))],
    out_specs=pl.BlockSpec((632, CNT_W), lambda i: (i, 0)),
    out_shape=jax.ShapeDtypeStruct((N_ACC, CNT_W), jnp.float32),
)


def _make_dense(apply_ln):
    return pl.pallas_call(
        functools.partial(_dense_body, apply_ln),
        grid=(N // BN,),
        in_specs=[
            pl.BlockSpec((NC, BN, D), lambda i: (0, i, 0)),
            pl.BlockSpec((BN, CNT_W), lambda i: (i, 0)),
            pl.BlockSpec((BN, D), lambda i: (i, 0)),
            pl.BlockSpec((D, D), lambda i: (0, 0)),
            pl.BlockSpec((1, D), lambda i: (0, 0)),
            pl.BlockSpec((D, D), lambda i: (0, 0)),
            pl.BlockSpec((1, D), lambda i: (0, 0)),
            pl.BlockSpec((1, D), lambda i: (0, 0)),
        ],
        out_specs=pl.BlockSpec((BN, D), lambda i: (i, 0)),
        out_shape=jax.ShapeDtypeStruct((N, D), jnp.float32),
    )


_dense_ln = _make_dense(True)
_dense_plain = _make_dense(False)


def kernel(x, edge_index, Wl0, Wl1, Wl2, Wl3, Wl4, bl0, bl1, bl2, bl3, bl4,
           Wr0, Wr1, Wr2, Wr3, Wr4, g0, g1, g2, g3, b0, b1, b2, b3):
    Wls = (Wl0, Wl1, Wl2, Wl3, Wl4)
    bls = (bl0, bl1, bl2, bl3, bl4)
    Wrs = (Wr0, Wr1, Wr2, Wr3, Wr4)
    gs = (g0, g1, g2, g3)
    bs = (b0, b1, b2, b3)

    src = edge_index[0]
    dst = edge_index[1]
    pad = E_PAD - E
    src_p = jnp.concatenate([src, jnp.zeros((pad,), jnp.int32)])
    dst_p = jnp.concatenate([dst, jnp.full((pad,), N, jnp.int32)])
    srcm = src_p.reshape(NW * CHUNKS, CH)
    dstm = dst_p.reshape(NW * CHUNKS, CH)
    zeros128 = jnp.zeros((N_ACC, D), jnp.float32)
    ones_chunk = jnp.ones((CH, D), jnp.float32)

    cnt = _narrow_call(_cnt_call(dstm, ones_chunk, zeros128))

    h = x
    for i in range(NUM_LAYERS):
        p = _agg_call(h, srcm, dstm, zeros128)
        dense = _dense_ln if i < NUM_LAYERS - 1 else _dense_plain
        gi = gs[i] if i < NUM_LAYERS - 1 else g0
        bi = bs[i] if i < NUM_LAYERS - 1 else b0
        h = dense(p, cnt, h, Wls[i], bls[i].reshape(1, D), Wrs[i],
                  gi.reshape(1, D), bi.reshape(1, D))
    return h


# speedup vs baseline: 1.4239x; 1.4239x over previous
import functools

import jax
import jax.numpy as jnp
from jax import lax
from jax.experimental import pallas as pl
from jax.experimental.pallas import tpu as pltpu
from jax.experimental.pallas import tpu_sc as plsc

N = 10000
D = 128
E = 320000
NUM_LAYERS = 5

NC = 2
NS = 16
NW = NC * NS
CH = 128
CHUNKS = 80
E_PAD = NW * CHUNKS * CH
N_ACC = 10112
ROWS_PER_TILE = N_ACC // NS
CNT_W = 16

_mesh = plsc.VectorSubcoreMesh(
    core_axis_name="c", subcore_axis_name="s", num_cores=NC, num_subcores=NS)


NBUF = 2
GIDX = 8
N0 = 128
N1 = 32
C0 = NS * N0


def _agg_body(h_hbm, srcm, dstm, zeros_hbm, p_hbm, acc,
              idx_v, rows0, rows1, isem, g0, g1, s0, s1):
    cid = lax.axis_index("c")
    sid = lax.axis_index("s")
    rows = (rows0, rows1)
    gsem = (g0, g1)
    ssem = (s0, s1)
    my_chunks = jnp.where(cid == 0, N0, N1)
    base_chunk = jnp.where(cid == 0, sid * N0, C0 + sid * N1)
    ngroups = my_chunks // GIDX

    @pl.when(ngroups > 0)
    def _prefetch0():
        pltpu.async_copy(srcm.at[pl.ds(base_chunk, GIDX)], idx_v.at[0, 0], isem)
        pltpu.async_copy(dstm.at[pl.ds(base_chunk, GIDX)], idx_v.at[0, 1], isem)
    pltpu.sync_copy(zeros_hbm.at[pl.ds(sid * ROWS_PER_TILE, ROWS_PER_TILE)],
                    acc.at[pl.ds(sid * ROWS_PER_TILE, ROWS_PER_TILE)])
    plsc.subcore_barrier()

    def group(g, carry):
        pb = g % 2
        pltpu.make_async_copy(srcm.at[pl.ds(base_chunk, GIDX)],
                              idx_v.at[0, 0], isem).wait()
        pltpu.make_async_copy(dstm.at[pl.ds(base_chunk, GIDX)],
                              idx_v.at[0, 1], isem).wait()

        @pl.when(g + 1 < ngroups)
        def _prefetch():
            nb = (g + 1) % 2
            off = base_chunk + (g + 1) * GIDX
            pltpu.async_copy(srcm.at[pl.ds(off, GIDX)], idx_v.at[nb, 0], isem)
            pltpu.async_copy(dstm.at[pl.ds(off, GIDX)], idx_v.at[nb, 1], isem)

        def step(j2, c2):
            base = j2 * NBUF
            gathers = [
                pltpu.async_copy(h_hbm.at[idx_v.at[pb, 0, base + b]],
                                 rows[b], gsem[b])
                for b in range(NBUF)
            ]
            scatters = []
            for b in range(NBUF):
                gathers[b].wait()
                scatters.append(
                    pltpu.async_copy(rows[b],
                                     acc.at[idx_v.at[pb, 1, base + b]],
                                     ssem[b], add=True))
            for b in range(NBUF):
                scatters[b].wait()
            return c2

        lax.fori_loop(0, GIDX // NBUF, step, 0)
        return carry

    lax.fori_loop(0, ngroups, group, 0)

    plsc.subcore_barrier()
    pltpu.sync_copy(acc.at[pl.ds(sid * ROWS_PER_TILE, ROWS_PER_TILE)],
                    p_hbm.at[cid, pl.ds(sid * ROWS_PER_TILE, ROWS_PER_TILE)])


_agg_call = pl.kernel(
    _agg_body,
    out_type=jax.ShapeDtypeStruct((NC, N_ACC, D), jnp.float32),
    mesh=_mesh,
    scratch_types=[
        pltpu.VMEM_SHARED((N_ACC, D), jnp.float32),
        pltpu.VMEM((2, 2, GIDX, CH), jnp.int32),
        pltpu.VMEM((CH, D), jnp.float32),
        pltpu.VMEM((CH, D), jnp.float32),
        pltpu.SemaphoreType.DMA,
        pltpu.SemaphoreType.DMA,
        pltpu.SemaphoreType.DMA,
        pltpu.SemaphoreType.DMA,
        pltpu.SemaphoreType.DMA,
    ],
)


def _cnt_body(dstm, ones_hbm, zeros_hbm, c_hbm, acc, dst_v, ones_v):
    cid = lax.axis_index("c")
    sid = lax.axis_index("s")
    w = sid * NC + cid
    pltpu.sync_copy(zeros_hbm.at[pl.ds(sid * ROWS_PER_TILE, ROWS_PER_TILE)],
                    acc.at[pl.ds(sid * ROWS_PER_TILE, ROWS_PER_TILE)])
    pltpu.sync_copy(ones_hbm, ones_v)
    pltpu.sync_copy(dstm.at[pl.ds(w * CHUNKS, CHUNKS)], dst_v)
    plsc.subcore_barrier()

    def step(j, carry):
        pltpu.sync_copy(ones_v, acc.at[dst_v.at[j]], add=True)
        return carry

    lax.fori_loop(0, CHUNKS, step, 0)
    plsc.subcore_barrier()
    pltpu.sync_copy(acc.at[pl.ds(sid * ROWS_PER_TILE, ROWS_PER_TILE)],
                    c_hbm.at[cid, pl.ds(sid * ROWS_PER_TILE, ROWS_PER_TILE)])


_cnt_call = pl.kernel(
    _cnt_body,
    out_type=jax.ShapeDtypeStruct((NC, N_ACC, D), jnp.float32),
    mesh=_mesh,
    scratch_types=[
        pltpu.VMEM_SHARED((N_ACC, D), jnp.float32),
        pltpu.VMEM((CHUNKS, CH), jnp.int32),
        pltpu.VMEM((CH, D), jnp.float32),
    ],
)


def _narrow_body(c_ref, o_ref):
    o_ref[...] = c_ref[0, :, :CNT_W] + c_ref[1, :, :CNT_W]


def _dense_body(apply_ln, p_ref, cnt_ref, h_ref, wl_ref, bl_ref, wr_ref,
                g_ref, b_ref, o_ref):
    p = p_ref[0] + p_ref[1]
    c = cnt_ref[:, 0:1]
    mean = p / jnp.maximum(c, 1.0)
    out = lax.dot_general(mean, wl_ref[...], (((1,), (1,)), ((), ())),
                          preferred_element_type=jnp.float32)
    out = out + bl_ref[...]
    out = out + lax.dot_general(h_ref[...], wr_ref[...], (((1,), (1,)), ((), ())),
                                preferred_element_type=jnp.float32)
    if apply_ln:
        mu = jnp.mean(out, axis=-1, keepdims=True)
        var = jnp.mean((out - mu) ** 2, axis=-1, keepdims=True)
        out = (out - mu) * lax.rsqrt(var + 1e-5) * g_ref[...] + b_ref[...]
        out = jnp.maximum(out, 0.0)
    o_ref[...] = out


BN = 400


_narrow_call = pl.pallas_call(
    _narrow_body,
    grid=(N_ACC // 632,),
    in_specs=[pl.BlockSpec((NC, 632, D), lambda i: (0, i, 0))],
    out_specs=pl.BlockSpec((632, CNT_W), lambda i: (i, 0)),
    out_shape=jax.ShapeDtypeStruct((N_ACC, CNT_W), jnp.float32),
)


def _make_dense(apply_ln):
    return pl.pallas_call(
        functools.partial(_dense_body, apply_ln),
        grid=(N // BN,),
        in_specs=[
            pl.BlockSpec((NC, BN, D), lambda i: (0, i, 0)),
            pl.BlockSpec((BN, CNT_W), lambda i: (i, 0)),
            pl.BlockSpec((BN, D), lambda i: (i, 0)),
            pl.BlockSpec((D, D), lambda i: (0, 0)),
            pl.BlockSpec((1, D), lambda i: (0, 0)),
            pl.BlockSpec((D, D), lambda i: (0, 0)),
            pl.BlockSpec((1, D), lambda i: (0, 0)),
            pl.BlockSpec((1, D), lambda i: (0, 0)),
        ],
        out_specs=pl.BlockSpec((BN, D), lambda i: (i, 0)),
        out_shape=jax.ShapeDtypeStruct((N, D), jnp.float32),
    )


_dense_ln = _make_dense(True)
_dense_plain = _make_dense(False)


def kernel(x, edge_index, Wl0, Wl1, Wl2, Wl3, Wl4, bl0, bl1, bl2, bl3, bl4,
           Wr0, Wr1, Wr2, Wr3, Wr4, g0, g1, g2, g3, b0, b1, b2, b3):
    Wls = (Wl0, Wl1, Wl2, Wl3, Wl4)
    bls = (bl0, bl1, bl2, bl3, bl4)
    Wrs = (Wr0, Wr1, Wr2, Wr3, Wr4)
    gs = (g0, g1, g2, g3)
    bs = (b0, b1, b2, b3)

    src = edge_index[0]
    dst = edge_index[1]
    pad = E_PAD - E
    src_p = jnp.concatenate([src, jnp.zeros((pad,), jnp.int32)])
    dst_p = jnp.concatenate([dst, jnp.full((pad,), N, jnp.int32)])
    srcm = src_p.reshape(NW * CHUNKS, CH)
    dstm = dst_p.reshape(NW * CHUNKS, CH)
    zeros128 = jnp.zeros((N_ACC, D), jnp.float32)
    ones_chunk = jnp.ones((CH, D), jnp.float32)

    cnt = _narrow_call(_cnt_call(dstm, ones_chunk, zeros128))

    h = x
    for i in range(NUM_LAYERS):
        p = _agg_call(h, srcm, dstm, zeros128)
        dense = _dense_ln if i < NUM_LAYERS - 1 else _dense_plain
        gi = gs[i] if i < NUM_LAYERS - 1 else g0
        bi = bs[i] if i < NUM_LAYERS - 1 else b0
        h = dense(p, cnt, h, Wls[i], bls[i].reshape(1, D), Wrs[i],
                  gi.reshape(1, D), bi.reshape(1, D))
    return h


# speedup vs baseline: 1.4368x; 1.0090x over previous
import functools

import jax
import jax.numpy as jnp
from jax import lax
from jax.experimental import pallas as pl
from jax.experimental.pallas import tpu as pltpu
from jax.experimental.pallas import tpu_sc as plsc

N = 10000
D = 128
E = 320000
NUM_LAYERS = 5

NC = 2
NS = 16
NW = NC * NS
CH = 128
CHUNKS = 80
E_PAD = NW * CHUNKS * CH
N_ACC = 10112
ROWS_PER_TILE = N_ACC // NS
CNT_W = 16

_mesh = plsc.VectorSubcoreMesh(
    core_axis_name="c", subcore_axis_name="s", num_cores=NC, num_subcores=NS)


NBUF = 2
GIDX = 8
N0 = 136
N1 = 24
C0 = NS * N0


def _agg_body(h_hbm, srcm, dstm, zeros_hbm, p_hbm, acc,
              idx_v, *bufs):
    rows = bufs[:NBUF]
    isem = bufs[NBUF]
    gsem = bufs[NBUF + 1:2 * NBUF + 1]
    ssem = bufs[2 * NBUF + 1:3 * NBUF + 1]
    cid = lax.axis_index("c")
    sid = lax.axis_index("s")
    my_chunks = jnp.where(cid == 0, N0, N1)
    base_chunk = jnp.where(cid == 0, sid * N0, C0 + sid * N1)
    ngroups = my_chunks // GIDX

    @pl.when(ngroups > 0)
    def _prefetch0():
        pltpu.async_copy(srcm.at[pl.ds(base_chunk, GIDX)], idx_v.at[0, 0], isem)
        pltpu.async_copy(dstm.at[pl.ds(base_chunk, GIDX)], idx_v.at[0, 1], isem)
    pltpu.sync_copy(zeros_hbm.at[pl.ds(sid * ROWS_PER_TILE, ROWS_PER_TILE)],
                    acc.at[pl.ds(sid * ROWS_PER_TILE, ROWS_PER_TILE)])
    plsc.subcore_barrier()

    def group(g, carry):
        pb = g % 2
        pltpu.make_async_copy(srcm.at[pl.ds(base_chunk, GIDX)],
                              idx_v.at[0, 0], isem).wait()
        pltpu.make_async_copy(dstm.at[pl.ds(base_chunk, GIDX)],
                              idx_v.at[0, 1], isem).wait()

        @pl.when(g + 1 < ngroups)
        def _prefetch():
            nb = (g + 1) % 2
            off = base_chunk + (g + 1) * GIDX
            pltpu.async_copy(srcm.at[pl.ds(off, GIDX)], idx_v.at[nb, 0], isem)
            pltpu.async_copy(dstm.at[pl.ds(off, GIDX)], idx_v.at[nb, 1], isem)

        def step(j2, c2):
            base = j2 * NBUF
            gathers = [
                pltpu.async_copy(h_hbm.at[idx_v.at[pb, 0, base + b]],
                                 rows[b], gsem[b])
                for b in range(NBUF)
            ]
            scatters = []
            for b in range(NBUF):
                gathers[b].wait()
                scatters.append(
                    pltpu.async_copy(rows[b],
                                     acc.at[idx_v.at[pb, 1, base + b]],
                                     ssem[b], add=True))
            for b in range(NBUF):
                scatters[b].wait()
            return c2

        lax.fori_loop(0, GIDX // NBUF, step, 0)
        return carry

    lax.fori_loop(0, ngroups, group, 0)

    plsc.subcore_barrier()
    pltpu.sync_copy(acc.at[pl.ds(sid * ROWS_PER_TILE, ROWS_PER_TILE)],
                    p_hbm.at[cid, pl.ds(sid * ROWS_PER_TILE, ROWS_PER_TILE)])


_agg_call = pl.kernel(
    _agg_body,
    out_type=jax.ShapeDtypeStruct((NC, N_ACC, D), jnp.float32),
    mesh=_mesh,
    scratch_types=(
        [pltpu.VMEM_SHARED((N_ACC, D), jnp.float32),
         pltpu.VMEM((2, 2, GIDX, CH), jnp.int32)]
        + [pltpu.VMEM((CH, D), jnp.float32) for _ in range(NBUF)]
        + [pltpu.SemaphoreType.DMA for _ in range(2 * NBUF + 1)]
    ),
)


def _cnt_body(dstm, ones_hbm, zeros_hbm, c_hbm, acc, dst_v, ones_v):
    cid = lax.axis_index("c")
    sid = lax.axis_index("s")
    w = sid * NC + cid
    pltpu.sync_copy(zeros_hbm.at[pl.ds(sid * ROWS_PER_TILE, ROWS_PER_TILE)],
                    acc.at[pl.ds(sid * ROWS_PER_TILE, ROWS_PER_TILE)])
    pltpu.sync_copy(ones_hbm, ones_v)
    pltpu.sync_copy(dstm.at[pl.ds(w * CHUNKS, CHUNKS)], dst_v)
    plsc.subcore_barrier()

    def step(j, carry):
        pltpu.sync_copy(ones_v, acc.at[dst_v.at[j]], add=True)
        return carry

    lax.fori_loop(0, CHUNKS, step, 0)
    plsc.subcore_barrier()
    pltpu.sync_copy(acc.at[pl.ds(sid * ROWS_PER_TILE, ROWS_PER_TILE)],
                    c_hbm.at[cid, pl.ds(sid * ROWS_PER_TILE, ROWS_PER_TILE)])


_cnt_call = pl.kernel(
    _cnt_body,
    out_type=jax.ShapeDtypeStruct((NC, N_ACC, D), jnp.float32),
    mesh=_mesh,
    scratch_types=[
        pltpu.VMEM_SHARED((N_ACC, D), jnp.float32),
        pltpu.VMEM((CHUNKS, CH), jnp.int32),
        pltpu.VMEM((CH, D), jnp.float32),
    ],
)


def _narrow_body(c_ref, o_ref):
    o_ref[...] = c_ref[0, :, :CNT_W] + c_ref[1, :, :CNT_W]


def _dense_body(apply_ln, p_ref, cnt_ref, h_ref, wl_ref, bl_ref, wr_ref,
                g_ref, b_ref, o_ref):
    p = p_ref[0] + p_ref[1]
    c = cnt_ref[:, 0:1]
    mean = p / jnp.maximum(c, 1.0)
    out = lax.dot_general(mean, wl_ref[...], (((1,), (1,)), ((), ())),
                          preferred_element_type=jnp.float32)
    out = out + bl_ref[...]
    out = out + lax.dot_general(h_ref[...], wr_ref[...], (((1,), (1,)), ((), ())),
                                preferred_element_type=jnp.float32)
    if apply_ln:
        mu = jnp.mean(out, axis=-1, keepdims=True)
        var = jnp.mean((out - mu) ** 2, axis=-1, keepdims=True)
        out = (out - mu) * lax.rsqrt(var + 1e-5) * g_ref[...] + b_ref[...]
        out = jnp.maximum(out, 0.0)
    o_ref[...] = out


BN = 400


_narrow_call = pl.pallas_call(
    _narrow_body,
    grid=(N_ACC // 632,),
    in_specs=[pl.BlockSpec((NC, 632, D), lambda i: (0, i, 0))],
    out_specs=pl.BlockSpec((632, CNT_W), lambda i: (i, 0)),
    out_shape=jax.ShapeDtypeStruct((N_ACC, CNT_W), jnp.float32),
)


def _make_dense(apply_ln):
    return pl.pallas_call(
        functools.partial(_dense_body, apply_ln),
        grid=(N // BN,),
        in_specs=[
            pl.BlockSpec((NC, BN, D), lambda i: (0, i, 0)),
            pl.BlockSpec((BN, CNT_W), lambda i: (i, 0)),
            pl.BlockSpec((BN, D), lambda i: (i, 0)),
            pl.BlockSpec((D, D), lambda i: (0, 0)),
            pl.BlockSpec((1, D), lambda i: (0, 0)),
            pl.BlockSpec((D, D), lambda i: (0, 0)),
            pl.BlockSpec((1, D), lambda i: (0, 0)),
            pl.BlockSpec((1, D), lambda i: (0, 0)),
        ],
        out_specs=pl.BlockSpec((BN, D), lambda i: (i, 0)),
        out_shape=jax.ShapeDtypeStruct((N, D), jnp.float32),
    )


_dense_ln = _make_dense(True)
_dense_plain = _make_dense(False)


def kernel(x, edge_index, Wl0, Wl1, Wl2, Wl3, Wl4, bl0, bl1, bl2, bl3, bl4,
           Wr0, Wr1, Wr2, Wr3, Wr4, g0, g1, g2, g3, b0, b1, b2, b3):
    Wls = (Wl0, Wl1, Wl2, Wl3, Wl4)
    bls = (bl0, bl1, bl2, bl3, bl4)
    Wrs = (Wr0, Wr1, Wr2, Wr3, Wr4)
    gs = (g0, g1, g2, g3)
    bs = (b0, b1, b2, b3)

    src = edge_index[0]
    dst = edge_index[1]
    pad = E_PAD - E
    src_p = jnp.concatenate([src, jnp.zeros((pad,), jnp.int32)])
    dst_p = jnp.concatenate([dst, jnp.full((pad,), N, jnp.int32)])
    srcm = src_p.reshape(NW * CHUNKS, CH)
    dstm = dst_p.reshape(NW * CHUNKS, CH)
    zeros128 = jnp.zeros((N_ACC, D), jnp.float32)
    ones_chunk = jnp.ones((CH, D), jnp.float32)

    cnt = _narrow_call(_cnt_call(dstm, ones_chunk, zeros128))

    h = x
    for i in range(NUM_LAYERS):
        p = _agg_call(h, srcm, dstm, zeros128)
        dense = _dense_ln if i < NUM_LAYERS - 1 else _dense_plain
        gi = gs[i] if i < NUM_LAYERS - 1 else g0
        bi = bs[i] if i < NUM_LAYERS - 1 else b0
        h = dense(p, cnt, h, Wls[i], bls[i].reshape(1, D), Wrs[i],
                  gi.reshape(1, D), bi.reshape(1, D))
    return h


# speedup vs baseline: 1.5063x; 1.0484x over previous
import functools

import jax
import jax.numpy as jnp
from jax import lax
from jax.experimental import pallas as pl
from jax.experimental.pallas import tpu as pltpu
from jax.experimental.pallas import tpu_sc as plsc

N = 10000
D = 128
E = 320000
NUM_LAYERS = 5

NC = 2
NS = 16
NW = NC * NS
CH = 128
CHUNKS = 80
E_PAD = NW * CHUNKS * CH
N_ACC = 10112
ROWS_PER_TILE = N_ACC // NS
CNT_W = 16

_mesh = plsc.VectorSubcoreMesh(
    core_axis_name="c", subcore_axis_name="s", num_cores=NC, num_subcores=NS)


NBUF = 2
GIDX = 8
N0 = 144
N1 = 16
C0 = NS * N0


def _agg_body(h_hbm, srcm, dstm, zeros_hbm, p_hbm, acc,
              idx_v, *bufs):
    rows = bufs[:NBUF]
    isem = bufs[NBUF]
    gsem = bufs[NBUF + 1:2 * NBUF + 1]
    ssem = bufs[2 * NBUF + 1:3 * NBUF + 1]
    cid = lax.axis_index("c")
    sid = lax.axis_index("s")
    my_chunks = jnp.where(cid == 0, N0, N1)
    base_chunk = jnp.where(cid == 0, sid * N0, C0 + sid * N1)
    ngroups = my_chunks // GIDX

    @pl.when(ngroups > 0)
    def _prefetch0():
        pltpu.async_copy(srcm.at[pl.ds(base_chunk, GIDX)], idx_v.at[0, 0], isem)
        pltpu.async_copy(dstm.at[pl.ds(base_chunk, GIDX)], idx_v.at[0, 1], isem)
    pltpu.sync_copy(zeros_hbm.at[pl.ds(sid * ROWS_PER_TILE, ROWS_PER_TILE)],
                    acc.at[pl.ds(sid * ROWS_PER_TILE, ROWS_PER_TILE)])
    plsc.subcore_barrier()

    def group(g, carry):
        pb = g % 2
        pltpu.make_async_copy(srcm.at[pl.ds(base_chunk, GIDX)],
                              idx_v.at[0, 0], isem).wait()
        pltpu.make_async_copy(dstm.at[pl.ds(base_chunk, GIDX)],
                              idx_v.at[0, 1], isem).wait()

        @pl.when(g + 1 < ngroups)
        def _prefetch():
            nb = (g + 1) % 2
            off = base_chunk + (g + 1) * GIDX
            pltpu.async_copy(srcm.at[pl.ds(off, GIDX)], idx_v.at[nb, 0], isem)
            pltpu.async_copy(dstm.at[pl.ds(off, GIDX)], idx_v.at[nb, 1], isem)

        def step(j2, c2):
            base = j2 * NBUF
            gathers = [
                pltpu.async_copy(h_hbm.at[idx_v.at[pb, 0, base + b]],
                                 rows[b], gsem[b])
                for b in range(NBUF)
            ]
            scatters = []
            for b in range(NBUF):
                gathers[b].wait()
                scatters.append(
                    pltpu.async_copy(rows[b],
                                     acc.at[idx_v.at[pb, 1, base + b]],
                                     ssem[b], add=True))
            for b in range(NBUF):
                scatters[b].wait()
            return c2

        lax.fori_loop(0, GIDX // NBUF, step, 0)
        return carry

    lax.fori_loop(0, ngroups, group, 0)

    plsc.subcore_barrier()
    pltpu.sync_copy(acc.at[pl.ds(sid * ROWS_PER_TILE, ROWS_PER_TILE)],
                    p_hbm.at[cid, pl.ds(sid * ROWS_PER_TILE, ROWS_PER_TILE)])


_agg_call = pl.kernel(
    _agg_body,
    out_type=jax.ShapeDtypeStruct((NC, N_ACC, D), jnp.float32),
    mesh=_mesh,
    scratch_types=(
        [pltpu.VMEM_SHARED((N_ACC, D), jnp.float32),
         pltpu.VMEM((2, 2, GIDX, CH), jnp.int32)]
        + [pltpu.VMEM((CH, D), jnp.float32) for _ in range(NBUF)]
        + [pltpu.SemaphoreType.DMA for _ in range(2 * NBUF + 1)]
    ),
)


def _cnt_body(dstm, ones_hbm, zeros_hbm, c_hbm, acc, dst_v, ones_v):
    cid = lax.axis_index("c")
    sid = lax.axis_index("s")
    w = sid * NC + cid
    pltpu.sync_copy(zeros_hbm.at[pl.ds(sid * ROWS_PER_TILE, ROWS_PER_TILE)],
                    acc.at[pl.ds(sid * ROWS_PER_TILE, ROWS_PER_TILE)])
    pltpu.sync_copy(ones_hbm, ones_v)
    pltpu.sync_copy(dstm.at[pl.ds(w * CHUNKS, CHUNKS)], dst_v)
    plsc.subcore_barrier()

    def step(j, carry):
        pltpu.sync_copy(ones_v, acc.at[dst_v.at[j]], add=True)
        return carry

    lax.fori_loop(0, CHUNKS, step, 0)
    plsc.subcore_barrier()
    pltpu.sync_copy(acc.at[pl.ds(sid * ROWS_PER_TILE, ROWS_PER_TILE)],
                    c_hbm.at[cid, pl.ds(sid * ROWS_PER_TILE, ROWS_PER_TILE)])


_cnt_call = pl.kernel(
    _cnt_body,
    out_type=jax.ShapeDtypeStruct((NC, N_ACC, D), jnp.float32),
    mesh=_mesh,
    scratch_types=[
        pltpu.VMEM_SHARED((N_ACC, D), jnp.float32),
        pltpu.VMEM((CHUNKS, CH), jnp.int32),
        pltpu.VMEM((CH, D), jnp.float32),
    ],
)


def _narrow_body(c_ref, o_ref):
    o_ref[...] = c_ref[0, :, :CNT_W] + c_ref[1, :, :CNT_W]


def _dense_body(apply_ln, p_ref, cnt_ref, h_ref, wl_ref, bl_ref, wr_ref,
                g_ref, b_ref, o_ref):
    p = p_ref[0] + p_ref[1]
    c = cnt_ref[:, 0:1]
    mean = p / jnp.maximum(c, 1.0)
    out = lax.dot_general(mean, wl_ref[...], (((1,), (1,)), ((), ())),
                          preferred_element_type=jnp.float32)
    out = out + bl_ref[...]
    out = out + lax.dot_general(h_ref[...], wr_ref[...], (((1,), (1,)), ((), ())),
                                preferred_element_type=jnp.float32)
    if apply_ln:
        mu = jnp.mean(out, axis=-1, keepdims=True)
        var = jnp.mean((out - mu) ** 2, axis=-1, keepdims=True)
        out = (out - mu) * lax.rsqrt(var + 1e-5) * g_ref[...] + b_ref[...]
        out = jnp.maximum(out, 0.0)
    o_ref[...] = out


BN = 400


_narrow_call = pl.pallas_call(
    _narrow_body,
    grid=(N_ACC // 632,),
    in_specs=[pl.BlockSpec((NC, 632, D), lambda i: (0, i, 0))],
    out_specs=pl.BlockSpec((632, CNT_W), lambda i: (i, 0)),
    out_shape=jax.ShapeDtypeStruct((N_ACC, CNT_W), jnp.float32),
)


def _make_dense(apply_ln):
    return pl.pallas_call(
        functools.partial(_dense_body, apply_ln),
        grid=(N // BN,),
        in_specs=[
            pl.BlockSpec((NC, BN, D), lambda i: (0, i, 0)),
            pl.BlockSpec((BN, CNT_W), lambda i: (i, 0)),
            pl.BlockSpec((BN, D), lambda i: (i, 0)),
            pl.BlockSpec((D, D), lambda i: (0, 0)),
            pl.BlockSpec((1, D), lambda i: (0, 0)),
            pl.BlockSpec((D, D), lambda i: (0, 0)),
            pl.BlockSpec((1, D), lambda i: (0, 0)),
            pl.BlockSpec((1, D), lambda i: (0, 0)),
        ],
        out_specs=pl.BlockSpec((BN, D), lambda i: (i, 0)),
        out_shape=jax.ShapeDtypeStruct((N, D), jnp.float32),
    )


_dense_ln = _make_dense(True)
_dense_plain = _make_dense(False)


def kernel(x, edge_index, Wl0, Wl1, Wl2, Wl3, Wl4, bl0, bl1, bl2, bl3, bl4,
           Wr0, Wr1, Wr2, Wr3, Wr4, g0, g1, g2, g3, b0, b1, b2, b3):
    Wls = (Wl0, Wl1, Wl2, Wl3, Wl4)
    bls = (bl0, bl1, bl2, bl3, bl4)
    Wrs = (Wr0, Wr1, Wr2, Wr3, Wr4)
    gs = (g0, g1, g2, g3)
    bs = (b0, b1, b2, b3)

    src = edge_index[0]
    dst = edge_index[1]
    pad = E_PAD - E
    src_p = jnp.concatenate([src, jnp.zeros((pad,), jnp.int32)])
    dst_p = jnp.concatenate([dst, jnp.full((pad,), N, jnp.int32)])
    srcm = src_p.reshape(NW * CHUNKS, CH)
    dstm = dst_p.reshape(NW * CHUNKS, CH)
    zeros128 = jnp.zeros((N_ACC, D), jnp.float32)
    ones_chunk = jnp.ones((CH, D), jnp.float32)

    cnt = _narrow_call(_cnt_call(dstm, ones_chunk, zeros128))

    h = x
    for i in range(NUM_LAYERS):
        p = _agg_call(h, srcm, dstm, zeros128)
        dense = _dense_ln if i < NUM_LAYERS - 1 else _dense_plain
        gi = gs[i] if i < NUM_LAYERS - 1 else g0
        bi = bs[i] if i < NUM_LAYERS - 1 else b0
        h = dense(p, cnt, h, Wls[i], bls[i].reshape(1, D), Wrs[i],
                  gi.reshape(1, D), bi.reshape(1, D))
    return h


# speedup vs baseline: 1.5151x; 1.0058x over previous
import functools

import jax
import jax.numpy as jnp
from jax import lax
from jax.experimental import pallas as pl
from jax.experimental.pallas import tpu as pltpu
from jax.experimental.pallas import tpu_sc as plsc

N = 10000
D = 128
E = 320000
NUM_LAYERS = 5

NC = 2
NS = 16
NW = NC * NS
CH = 128
CHUNKS = 80
E_PAD = NW * CHUNKS * CH
N_ACC = 10112
ROWS_PER_TILE = N_ACC // NS
CNT_W = 16

_mesh = plsc.VectorSubcoreMesh(
    core_axis_name="c", subcore_axis_name="s", num_cores=NC, num_subcores=NS)


NBUF = 2
GIDX = 8
N0 = 152
N1 = 8
C0 = NS * N0


def _agg_body(h_hbm, srcm, dstm, zeros_hbm, p_hbm, acc,
              idx_v, *bufs):
    rows = bufs[:NBUF]
    isem = bufs[NBUF]
    gsem = bufs[NBUF + 1:2 * NBUF + 1]
    ssem = bufs[2 * NBUF + 1:3 * NBUF + 1]
    cid = lax.axis_index("c")
    sid = lax.axis_index("s")
    my_chunks = jnp.where(cid == 0, N0, N1)
    base_chunk = jnp.where(cid == 0, sid * N0, C0 + sid * N1)
    ngroups = my_chunks // GIDX

    @pl.when(ngroups > 0)
    def _prefetch0():
        pltpu.async_copy(srcm.at[pl.ds(base_chunk, GIDX)], idx_v.at[0, 0], isem)
        pltpu.async_copy(dstm.at[pl.ds(base_chunk, GIDX)], idx_v.at[0, 1], isem)
    pltpu.sync_copy(zeros_hbm.at[pl.ds(sid * ROWS_PER_TILE, ROWS_PER_TILE)],
                    acc.at[pl.ds(sid * ROWS_PER_TILE, ROWS_PER_TILE)])
    plsc.subcore_barrier()

    def group(g, carry):
        pb = g % 2
        pltpu.make_async_copy(srcm.at[pl.ds(base_chunk, GIDX)],
                              idx_v.at[0, 0], isem).wait()
        pltpu.make_async_copy(dstm.at[pl.ds(base_chunk, GIDX)],
                              idx_v.at[0, 1], isem).wait()

        @pl.when(g + 1 < ngroups)
        def _prefetch():
            nb = (g + 1) % 2
            off = base_chunk + (g + 1) * GIDX
            pltpu.async_copy(srcm.at[pl.ds(off, GIDX)], idx_v.at[nb, 0], isem)
            pltpu.async_copy(dstm.at[pl.ds(off, GIDX)], idx_v.at[nb, 1], isem)

        def step(j2, c2):
            base = j2 * NBUF
            gathers = [
                pltpu.async_copy(h_hbm.at[idx_v.at[pb, 0, base + b]],
                                 rows[b], gsem[b])
                for b in range(NBUF)
            ]
            scatters = []
            for b in range(NBUF):
                gathers[b].wait()
                scatters.append(
                    pltpu.async_copy(rows[b],
                                     acc.at[idx_v.at[pb, 1, base + b]],
                                     ssem[b], add=True))
            for b in range(NBUF):
                scatters[b].wait()
            return c2

        lax.fori_loop(0, GIDX // NBUF, step, 0)
        return carry

    lax.fori_loop(0, ngroups, group, 0)

    plsc.subcore_barrier()
    pltpu.sync_copy(acc.at[pl.ds(sid * ROWS_PER_TILE, ROWS_PER_TILE)],
                    p_hbm.at[cid, pl.ds(sid * ROWS_PER_TILE, ROWS_PER_TILE)])


_agg_call = pl.kernel(
    _agg_body,
    out_type=jax.ShapeDtypeStruct((NC, N_ACC, D), jnp.float32),
    mesh=_mesh,
    scratch_types=(
        [pltpu.VMEM_SHARED((N_ACC, D), jnp.float32),
         pltpu.VMEM((2, 2, GIDX, CH), jnp.int32)]
        + [pltpu.VMEM((CH, D), jnp.float32) for _ in range(NBUF)]
        + [pltpu.SemaphoreType.DMA for _ in range(2 * NBUF + 1)]
    ),
)


def _cnt_body(dstm, ones_hbm, zeros_hbm, c_hbm, acc, dst_v, ones_v):
    cid = lax.axis_index("c")
    sid = lax.axis_index("s")
    w = sid * NC + cid
    pltpu.sync_copy(zeros_hbm.at[pl.ds(sid * ROWS_PER_TILE, ROWS_PER_TILE)],
                    acc.at[pl.ds(sid * ROWS_PER_TILE, ROWS_PER_TILE)])
    pltpu.sync_copy(ones_hbm, ones_v)
    pltpu.sync_copy(dstm.at[pl.ds(w * CHUNKS, CHUNKS)], dst_v)
    plsc.subcore_barrier()

    def step(j, carry):
        pltpu.sync_copy(ones_v, acc.at[dst_v.at[j]], add=True)
        return carry

    lax.fori_loop(0, CHUNKS, step, 0)
    plsc.subcore_barrier()
    pltpu.sync_copy(acc.at[pl.ds(sid * ROWS_PER_TILE, ROWS_PER_TILE)],
                    c_hbm.at[cid, pl.ds(sid * ROWS_PER_TILE, ROWS_PER_TILE)])


_cnt_call = pl.kernel(
    _cnt_body,
    out_type=jax.ShapeDtypeStruct((NC, N_ACC, D), jnp.float32),
    mesh=_mesh,
    scratch_types=[
        pltpu.VMEM_SHARED((N_ACC, D), jnp.float32),
        pltpu.VMEM((CHUNKS, CH), jnp.int32),
        pltpu.VMEM((CH, D), jnp.float32),
    ],
)


def _narrow_body(c_ref, o_ref):
    o_ref[...] = c_ref[0, :, :CNT_W] + c_ref[1, :, :CNT_W]


def _dense_body(apply_ln, p_ref, cnt_ref, h_ref, wl_ref, bl_ref, wr_ref,
                g_ref, b_ref, o_ref):
    p = p_ref[0] + p_ref[1]
    c = cnt_ref[:, 0:1]
    mean = p / jnp.maximum(c, 1.0)
    out = lax.dot_general(mean, wl_ref[...], (((1,), (1,)), ((), ())),
                          preferred_element_type=jnp.float32)
    out = out + bl_ref[...]
    out = out + lax.dot_general(h_ref[...], wr_ref[...], (((1,), (1,)), ((), ())),
                                preferred_element_type=jnp.float32)
    if apply_ln:
        mu = jnp.mean(out, axis=-1, keepdims=True)
        var = jnp.mean((out - mu) ** 2, axis=-1, keepdims=True)
        out = (out - mu) * lax.rsqrt(var + 1e-5) * g_ref[...] + b_ref[...]
        out = jnp.maximum(out, 0.0)
    o_ref[...] = out


BN = 400


_narrow_call = pl.pallas_call(
    _narrow_body,
    grid=(N_ACC // 632,),
    in_specs=[pl.BlockSpec((NC, 632, D), lambda i: (0, i, 0))],
    out_specs=pl.BlockSpec((632, CNT_W), lambda i: (i, 0)),
    out_shape=jax.ShapeDtypeStruct((N_ACC, CNT_W), jnp.float32),
)


def _make_dense(apply_ln):
    return pl.pallas_call(
        functools.partial(_dense_body, apply_ln),
        grid=(N // BN,),
        in_specs=[
            pl.BlockSpec((NC, BN, D), lambda i: (0, i, 0)),
            pl.BlockSpec((BN, CNT_W), lambda i: (i, 0)),
            pl.BlockSpec((BN, D), lambda i: (i, 0)),
            pl.BlockSpec((D, D), lambda i: (0, 0)),
            pl.BlockSpec((1, D), lambda i: (0, 0)),
            pl.BlockSpec((D, D), lambda i: (0, 0)),
            pl.BlockSpec((1, D), lambda i: (0, 0)),
            pl.BlockSpec((1, D), lambda i: (0, 0)),
        ],
        out_specs=pl.BlockSpec((BN, D), lambda i: (i, 0)),
        out_shape=jax.ShapeDtypeStruct((N, D), jnp.float32),
    )


_dense_ln = _make_dense(True)
_dense_plain = _make_dense(False)


def kernel(x, edge_index, Wl0, Wl1, Wl2, Wl3, Wl4, bl0, bl1, bl2, bl3, bl4,
           Wr0, Wr1, Wr2, Wr3, Wr4, g0, g1, g2, g3, b0, b1, b2, b3):
    Wls = (Wl0, Wl1, Wl2, Wl3, Wl4)
    bls = (bl0, bl1, bl2, bl3, bl4)
    Wrs = (Wr0, Wr1, Wr2, Wr3, Wr4)
    gs = (g0, g1, g2, g3)
    bs = (b0, b1, b2, b3)

    src = edge_index[0]
    dst = edge_index[1]
    pad = E_PAD - E
    src_p = jnp.concatenate([src, jnp.zeros((pad,), jnp.int32)])
    dst_p = jnp.concatenate([dst, jnp.full((pad,), N, jnp.int32)])
    srcm = src_p.reshape(NW * CHUNKS, CH)
    dstm = dst_p.reshape(NW * CHUNKS, CH)
    zeros128 = jnp.zeros((N_ACC, D), jnp.float32)
    ones_chunk = jnp.ones((CH, D), jnp.float32)

    cnt = _narrow_call(_cnt_call(dstm, ones_chunk, zeros128))

    h = x
    for i in range(NUM_LAYERS):
        p = _agg_call(h, srcm, dstm, zeros128)
        dense = _dense_ln if i < NUM_LAYERS - 1 else _dense_plain
        gi = gs[i] if i < NUM_LAYERS - 1 else g0
        bi = bs[i] if i < NUM_LAYERS - 1 else b0
        h = dense(p, cnt, h, Wls[i], bls[i].reshape(1, D), Wrs[i],
                  gi.reshape(1, D), bi.reshape(1, D))
    return h
